# baseline (speedup 1.0000x reference)
import functools

import jax
import jax.numpy as jnp
from jax import lax
from jax.experimental import pallas as pl
from jax.experimental.pallas import tpu as pltpu
from jax.experimental.pallas import tpu_sc as plsc

N = 2000
P = 4000
Pp = 4096
H = 512
OBJ = 4096
EPD = 384
NCLSP = 256
TN = 400
GN = 5
TP = 512
GP = 8
F32 = jnp.float32
NEG = -1e30
HI = lax.Precision.HIGHEST
NW = 32
BPW = Pp // NW
Np = 2048
NSTRIPE = Np // 16


def _dot(a, b):
    return lax.dot_general(a, b, (((1,), (0,)), ((), ())),
                           precision=lax.Precision.DEFAULT,
                           preferred_element_type=F32)


def _ln(x, g, b):
    m = jnp.mean(x, axis=-1, keepdims=True)
    v = jnp.mean((x - m) ** 2, axis=-1, keepdims=True)
    return (x - m) / jnp.sqrt(v + 1e-5) * g + b


def _dotbf(a, b):
    return lax.dot_general(a.astype(jnp.bfloat16), b.astype(jnp.bfloat16),
                           (((1,), (0,)), ((), ())),
                           preferred_element_type=F32)


def _merge_body(roi_ref, ep_ref, w1_ref, w2_ref, b_ref, out_ref):
    acc = _dotbf(roi_ref[...], w1_ref[...])
    acc += _dotbf(ep_ref[...], w2_ref[...])
    out_ref[...] = acc + b_ref[...]


def _phr_body(u_ref, w_ref, b_ref, out_ref):
    out_ref[...] = _dotbf(u_ref[...], w_ref[...]) + b_ref[...]


def _sov_body(obj_ref, ws_ref, wsb_ref, wo_ref, wob_ref, cw_ref, cb_ref,
              g_ref, bb_ref, s_ref, o_ref, conv_ref, csum_ref):
    obj = obj_ref[...]
    s_ref[...] = _dotbf(obj, ws_ref[...]) + wsb_ref[...]
    o_ref[...] = _dotbf(obj, wo_ref[...]) + wob_ref[...]
    conv = jnp.maximum(_dotbf(_ln(obj, g_ref[...], bb_ref[...]), cw_ref[...])
                       + cb_ref[...], 0.0)
    conv_ref[...] = conv
    csum_ref[...] = jnp.broadcast_to(jnp.sum(conv, axis=0, keepdims=True),
                                     (8, H))


def _sc_gather_a(s, o, conv, phr, wrow, p0, p1):
    CH = BPW // 2

    @functools.partial(
        pl.kernel,
        out_type=[jax.ShapeDtypeStruct((Pp, 16), F32),
                  jax.ShapeDtypeStruct((Pp, H), F32)],
        mesh=plsc.VectorSubcoreMesh(core_axis_name="c", subcore_axis_name="s"),
        scratch_types=[pltpu.VMEM((BPW,), jnp.int32),
                       pltpu.VMEM((BPW,), jnp.int32),
                       pltpu.VMEM((H,), F32),
                       pltpu.VMEM((CH, H), F32),
                       pltpu.VMEM((CH, H), F32),
                       pltpu.VMEM((CH, H), F32),
                       pltpu.VMEM((CH, 16), F32),
                       pltpu.SemaphoreType.DMA,
                       pltpu.SemaphoreType.DMA,
                       pltpu.SemaphoreType.DMA],
    )
    def k(s_hbm, o_hbm, c_hbm, phr_hbm, w_hbm, p0_hbm, p1_hbm,
          a_hbm, cg_hbm, i0_v, i1_v, wv, sgb, ogb, phb, avb, s0, s1, s2):
        wid = lax.axis_index("s") * 2 + lax.axis_index("c")
        base = wid * BPW
        pltpu.sync_copy(p0_hbm.at[pl.ds(base, BPW)], i0_v)
        pltpu.sync_copy(p1_hbm.at[pl.ds(base, BPW)], i1_v)
        pltpu.sync_copy(w_hbm, wv)
        for ch in range(2):
            g1 = pltpu.async_copy(
                s_hbm.at[i0_v.at[pl.ds(ch * CH, CH)]], sgb, s0)
            g2 = pltpu.async_copy(
                o_hbm.at[i1_v.at[pl.ds(ch * CH, CH)]], ogb, s1)
            g3 = pltpu.async_copy(
                phr_hbm.at[pl.ds(base + ch * CH, CH)], phb, s2)
            g1.wait()
            g2.wait()
            g3.wait()

            def body(r, carry):
                acc = jnp.zeros((16,), F32)
                for cc in range(H // 16):
                    hs = pl.ds(cc * 16, 16)
                    acc = acc + sgb[r, hs] * ogb[r, hs] * phb[r, hs] * wv[hs]
                avb[r, :] = acc
                return carry

            lax.fori_loop(0, CH, body, 0)
            pltpu.sync_copy(avb, a_hbm.at[pl.ds(base + ch * CH, CH)])
            g4 = pltpu.async_copy(
                c_hbm.at[i1_v.at[pl.ds(ch * CH, CH)]], sgb, s0)
            g4.wait()
            pltpu.sync_copy(sgb, cg_hbm.at[pl.ds(base + ch * CH, CH)])

    return k(s, o, conv, phr, wrow, p0, p1)


def _scatter_body(p0_ref, wcg_ref, out_ref):
    i = pl.program_id(0)
    rows = i * TN + lax.broadcasted_iota(jnp.int32, (TN, Pp), 0)
    p0 = p0_ref[0, :]
    g = (p0[None, :] == rows).astype(F32)
    out_ref[...] = _dot(g, wcg_ref[...])


def _paircoef_body(key_ref, a16_ref, wb_ref, offd_ref, cg_ref,
                   c_ref, d_ref, wcg_ref):
    i = pl.program_id(0)
    sl = pl.ds(i * TP, TP)
    kt = key_ref[0, sl]
    kf = key_ref[0, :]
    av = jnp.sum(a16_ref[...], axis=1) + wb_ref[0, 0]
    mm = kt[:, None] == kf[None, :]
    b = jnp.sum(jnp.where(mm, av[None, :], 0.0), axis=1)
    cnt = jnp.sum(mm.astype(F32), axis=1)
    invc = 1.0 / cnt
    offd = offd_ref[0, sl]
    c = offd * jnp.exp(b) * invc
    d = offd * invc
    c_ref[0, :] = c
    d_ref[0, :] = d
    wcg_ref[...] = cg_ref[...] * (c - d)[:, None]


def _ctx_body(obj_ref, conv_ref, c_ref, d_ref, p0_ref, wcg_ref,
              csum_ref, g2_ref, b2_ref, w1_ref, b1_ref, w2_ref, bb2_ref,
              out_ref):
    i = pl.program_id(0)
    rows = i * TN + lax.broadcasted_iota(jnp.int32, (TN, Pp), 0)
    p0 = p0_ref[0, :]
    rowmask = p0[None, :] == rows
    e = jnp.sum(jnp.where(rowmask, c_ref[0, :][None, :], 0.0), axis=1)
    dd = jnp.sum(jnp.where(rowmask, d_ref[0, :][None, :], 0.0), axis=1)
    ctxnum = _dotbf(rowmask.astype(F32), wcg_ref[...])
    z = ((N - 1) - dd) + e
    s_all = jnp.sum(csum_ref[...], axis=0, keepdims=True) * 0.125
    conv = conv_ref[...]
    context = ((s_all - conv) + ctxnum) / z[:, None]
    outputs = obj_ref[...] + context
    h = _ln(outputs, g2_ref[...], b2_ref[...])
    t = _dotbf(jnp.maximum(_dotbf(h, w1_ref[...]) + b1_ref[...], 0.0),
               w2_ref[...]) + bb2_ref[...]
    out_ref[...] = jnp.maximum(outputs + t, 0.0)


def _logits_body(obj_ref, g_ref, b_ref, w_ref, cb_ref, out_ref):
    out_ref[...] = _dot(_ln(obj_ref[...], g_ref[...], b_ref[...]),
                        w_ref[...]) + cb_ref[...]


def _row2d(v):
    return v.reshape(1, -1)


def _call(body, grid, in_specs, out_specs, out_shape, *args):
    return pl.pallas_call(
        body, grid=grid, in_specs=in_specs, out_specs=out_specs,
        out_shape=out_shape)(*args)


def kernel(roi_features, union_features, pair_idxs, obj_labels, obj_embed, pos_embed, merge_W, merge_b, phr_W, phr_b, ws_W_0, ws_b_0, wo_W_0, wo_b_0, w_W_0, w_b_0, conv_W_0, conv_b_0, trans_W1_0, trans_b1_0, trans_W2_0, trans_b2_0, ln1_g_0, ln1_b_0, ln2_g_0, ln2_b_0, ws_W_1, ws_b_1, wo_W_1, wo_b_1, w_W_1, w_b_1, conv_W_1, conv_b_1, trans_W1_1, trans_b1_1, trans_W2_1, trans_b2_1, ln1_g_1, ln1_b_1, ln2_g_1, ln2_b_1, ln_g, ln_b, cls_W, cls_b):
    f = F32
    ep = jnp.concatenate([obj_embed, pos_embed,
                          jnp.zeros((N, EPD - 328), f)], axis=1)
    w_ep = jnp.concatenate([merge_W[OBJ:, :],
                            jnp.zeros((EPD - 328, H), f)], axis=0)
    p0 = pair_idxs[:, 0].astype(jnp.int32)
    p1 = pair_idxs[:, 1].astype(jnp.int32)
    p0p = jnp.concatenate([p0, jnp.zeros((Pp - P,), jnp.int32)])
    p1p = jnp.concatenate([p1, jnp.zeros((Pp - P,), jnp.int32)])
    valid = (jnp.arange(Pp) < P)
    key = jnp.where(valid, p0p * N + p1p, N * N + jnp.arange(Pp)).astype(jnp.int32)
    offd = ((p0p != p1p) & valid).astype(f)
    p0r, keyr, offdr = map(_row2d, (p0p, key, offd))
    unionp = jnp.concatenate([union_features, jnp.zeros((Pp - P, OBJ), f)], axis=0)

    full = lambda shp: pl.BlockSpec(shp, lambda i: (0,) * len(shp))
    rowN = pl.BlockSpec((TN, H), lambda i: (i, 0))
    rowP = pl.BlockSpec((TP, H), lambda i: (i, 0))
    colN = pl.BlockSpec((TN, 1), lambda i: (i, 0))
    vecP = pl.BlockSpec((1, TP), lambda i: (0, i))

    obj = _call(_merge_body, (GN,),
                [pl.BlockSpec((TN, OBJ), lambda i: (i, 0)),
                 pl.BlockSpec((TN, EPD), lambda i: (i, 0)),
                 full((OBJ, H)), full((EPD, H)), full((1, H))],
                rowN, jax.ShapeDtypeStruct((N, H), f),
                roi_features, ep, merge_W[:OBJ, :], w_ep, _row2d(merge_b))

    phr = _call(_phr_body, (GP,),
                [pl.BlockSpec((TP, OBJ), lambda i: (i, 0)),
                 full((OBJ, H)), full((1, H))],
                rowP, jax.ShapeDtypeStruct((Pp, H), f),
                unionp, phr_W, _row2d(phr_b))

    layers = [
        (ws_W_0, ws_b_0, wo_W_0, wo_b_0, w_W_0, w_b_0, conv_W_0, conv_b_0,
         trans_W1_0, trans_b1_0, trans_W2_0, trans_b2_0, ln1_g_0, ln1_b_0,
         ln2_g_0, ln2_b_0),
        (ws_W_1, ws_b_1, wo_W_1, wo_b_1, w_W_1, w_b_1, conv_W_1, conv_b_1,
         trans_W1_1, trans_b1_1, trans_W2_1, trans_b2_1, ln1_g_1, ln1_b_1,
         ln2_g_1, ln2_b_1),
    ]

    for (ws_W, ws_b, wo_W, wo_b, w_W, w_b, conv_W, conv_b, tW1, tb1, tW2,
         tb2, g1, b1, g2, b2) in layers:
        s, o, conv, csum = _call(
            _sov_body, (GN,),
            [rowN, full((H, H)), full((1, H)), full((H, H)), full((1, H)),
             full((H, H)), full((1, H)), full((1, H)), full((1, H))],
            [rowN, rowN, rowN, pl.BlockSpec((8, H), lambda i: (i, 0))],
            [jax.ShapeDtypeStruct((N, H), f)] * 3
            + [jax.ShapeDtypeStruct((GN * 8, H), f)],
            obj, ws_W, _row2d(ws_b), wo_W, _row2d(wo_b), conv_W,
            _row2d(conv_b), _row2d(g1), _row2d(b1))

        a16, cg = _sc_gather_a(s, o, conv, phr, w_W[:, 0], p0p, p1p)

        c, dv, wcg = _call(_paircoef_body, (GP,),
                           [full((1, Pp)), full((Pp, 16)), full((1, 128)),
                            full((1, Pp)), rowP],
                           [vecP, vecP, rowP],
                           [jax.ShapeDtypeStruct((1, Pp), f)] * 2
                           + [jax.ShapeDtypeStruct((Pp, H), f)],
                           keyr, a16,
                           jnp.broadcast_to(w_b.reshape(1, 1), (1, 128)),
                           offdr, cg)

        obj = _call(_ctx_body, (GN,),
                    [rowN, rowN, full((1, Pp)), full((1, Pp)),
                     full((1, Pp)), full((Pp, H)), full((GN * 8, H)),
                     full((1, H)), full((1, H)), full((H, 2 * H)),
                     full((1, 2 * H)), full((2 * H, H)), full((1, H))],
                    rowN, jax.ShapeDtypeStruct((N, H), f),
                    obj, conv, c, dv, p0r, wcg, csum,
                    _row2d(g2), _row2d(b2), tW1, _row2d(tb1), tW2,
                    _row2d(tb2))

    cls_Wp = jnp.concatenate([cls_W, jnp.zeros((H, NCLSP - 151), f)], axis=1)
    cls_bp = jnp.concatenate([cls_b, jnp.zeros((NCLSP - 151,), f)])
    logits = _call(_logits_body, (GN,),
                   [rowN, full((1, H)), full((1, H)), full((H, NCLSP)),
                    full((1, NCLSP))],
                   pl.BlockSpec((TN, NCLSP), lambda i: (i, 0)),
                   jax.ShapeDtypeStruct((N, NCLSP), f),
                   obj, _row2d(ln_g), _row2d(ln_b), cls_Wp, _row2d(cls_bp))
    return logits[:, :151]


# speedup vs baseline: 1.0007x; 1.0007x over previous
import functools

import jax
import jax.numpy as jnp
from jax import lax
from jax.experimental import pallas as pl
from jax.experimental.pallas import tpu as pltpu
from jax.experimental.pallas import tpu_sc as plsc

N = 2000
P = 4000
Pp = 4096
H = 512
OBJ = 4096
EPD = 384
NCLSP = 256
TN = 400
GN = 5
TP = 512
GP = 8
F32 = jnp.float32
NW = 32
BPW = Pp // NW


def _dot(a, b):
    return lax.dot_general(a, b, (((1,), (0,)), ((), ())),
                           precision=lax.Precision.DEFAULT,
                           preferred_element_type=F32)


def _ln(x, g, b):
    m = jnp.mean(x, axis=-1, keepdims=True)
    v = jnp.mean((x - m) ** 2, axis=-1, keepdims=True)
    return (x - m) / jnp.sqrt(v + 1e-5) * g + b


def _dotbf(a, b):
    return lax.dot_general(a.astype(jnp.bfloat16), b.astype(jnp.bfloat16),
                           (((1,), (0,)), ((), ())),
                           preferred_element_type=F32)


def _merge_body(roi_ref, ep_ref, w1_ref, w2_ref, b_ref, out_ref):
    acc = _dotbf(roi_ref[...], w1_ref[...])
    acc += _dotbf(ep_ref[...], w2_ref[...])
    out_ref[...] = acc + b_ref[...]


def _phr_body(u_ref, w_ref, b_ref, out_ref):
    out_ref[...] = _dotbf(u_ref[...], w_ref[...]) + b_ref[...]


def _sov_body(obj_ref, ws_ref, wsb_ref, wo_ref, wob_ref, cw_ref, cb_ref,
              g_ref, bb_ref, s_ref, o_ref, conv_ref, csum_ref):
    obj = obj_ref[...]
    s_ref[...] = _dotbf(obj, ws_ref[...]) + wsb_ref[...]
    o_ref[...] = _dotbf(obj, wo_ref[...]) + wob_ref[...]
    conv = jnp.maximum(_dotbf(_ln(obj, g_ref[...], bb_ref[...]), cw_ref[...])
                       + cb_ref[...], 0.0)
    conv_ref[...] = conv
    csum_ref[...] = jnp.broadcast_to(jnp.sum(conv, axis=0, keepdims=True),
                                     (8, H))


def _sc_gather_a(s, o, conv, phr, wrow, p0, p1):
    CH = BPW // 2

    @functools.partial(
        pl.kernel,
        out_type=[jax.ShapeDtypeStruct((Pp, 16), F32),
                  jax.ShapeDtypeStruct((Pp, H), F32)],
        mesh=plsc.VectorSubcoreMesh(core_axis_name="c", subcore_axis_name="s"),
        scratch_types=[pltpu.VMEM((BPW,), jnp.int32),
                       pltpu.VMEM((BPW,), jnp.int32),
                       pltpu.VMEM((H,), F32),
                       pltpu.VMEM((CH, H), F32),
                       pltpu.VMEM((CH, H), F32),
                       pltpu.VMEM((CH, H), F32),
                       pltpu.VMEM((CH, 16), F32),
                       pltpu.SemaphoreType.DMA,
                       pltpu.SemaphoreType.DMA,
                       pltpu.SemaphoreType.DMA],
    )
    def k(s_hbm, o_hbm, c_hbm, phr_hbm, w_hbm, p0_hbm, p1_hbm,
          a_hbm, cg_hbm, i0_v, i1_v, wv, sgb, ogb, phb, avb, s0, s1, s2):
        wid = lax.axis_index("s") * 2 + lax.axis_index("c")
        base = wid * BPW
        pltpu.sync_copy(p0_hbm.at[pl.ds(base, BPW)], i0_v)
        pltpu.sync_copy(p1_hbm.at[pl.ds(base, BPW)], i1_v)
        pltpu.sync_copy(w_hbm, wv)
        for ch in range(2):
            g1 = pltpu.async_copy(
                s_hbm.at[i0_v.at[pl.ds(ch * CH, CH)]], sgb, s0)
            g2 = pltpu.async_copy(
                o_hbm.at[i1_v.at[pl.ds(ch * CH, CH)]], ogb, s1)
            g3 = pltpu.async_copy(
                phr_hbm.at[pl.ds(base + ch * CH, CH)], phb, s2)
            g1.wait()
            g2.wait()
            g3.wait()

            def body(r, carry):
                acc = jnp.zeros((16,), F32)
                for cc in range(H // 16):
                    hs = pl.ds(cc * 16, 16)
                    acc = acc + sgb[r, hs] * ogb[r, hs] * phb[r, hs] * wv[hs]
                avb[r, :] = acc
                return carry

            lax.fori_loop(0, CH, body, 0)
            pltpu.sync_copy(avb, a_hbm.at[pl.ds(base + ch * CH, CH)])
            g4 = pltpu.async_copy(
                c_hbm.at[i1_v.at[pl.ds(ch * CH, CH)]], sgb, s0)
            g4.wait()
            pltpu.sync_copy(sgb, cg_hbm.at[pl.ds(base + ch * CH, CH)])

    return k(s, o, conv, phr, wrow, p0, p1)


def _paircoef_body(key_ref, a16_ref, wb_ref, offd_ref, cg_ref,
                   c_ref, d_ref, wcg_ref):
    i = pl.program_id(0)
    sl = pl.ds(i * TP, TP)
    kt = key_ref[0, sl]
    kf = key_ref[0, :]
    av = jnp.sum(a16_ref[...], axis=1) + wb_ref[0, 0]
    mm = kt[:, None] == kf[None, :]
    b = jnp.sum(jnp.where(mm, av[None, :], 0.0), axis=1)
    cnt = jnp.sum(mm.astype(F32), axis=1)
    invc = 1.0 / cnt
    offd = offd_ref[0, sl]
    c = offd * jnp.exp(b) * invc
    d = offd * invc
    c_ref[0, :] = c
    d_ref[0, :] = d
    wcg_ref[...] = cg_ref[...] * (c - d)[:, None]


def _ctx_body(obj_ref, conv_ref, c_ref, d_ref, p0_ref, wcg_ref,
              csum_ref, g2_ref, b2_ref, w1_ref, b1_ref, w2_ref, bb2_ref,
              out_ref):
    i = pl.program_id(0)
    rows = i * TN + lax.broadcasted_iota(jnp.int32, (TN, Pp), 0)
    p0 = p0_ref[0, :]
    rowmask = p0[None, :] == rows
    e = jnp.sum(jnp.where(rowmask, c_ref[0, :][None, :], 0.0), axis=1)
    dd = jnp.sum(jnp.where(rowmask, d_ref[0, :][None, :], 0.0), axis=1)
    ctxnum = _dotbf(rowmask.astype(F32), wcg_ref[...])
    z = ((N - 1) - dd) + e
    s_all = jnp.sum(csum_ref[...], axis=0, keepdims=True) * 0.125
    conv = conv_ref[...]
    context = ((s_all - conv) + ctxnum) / z[:, None]
    outputs = obj_ref[...] + context
    h = _ln(outputs, g2_ref[...], b2_ref[...])
    t = _dotbf(jnp.maximum(_dotbf(h, w1_ref[...]) + b1_ref[...], 0.0),
               w2_ref[...]) + bb2_ref[...]
    out_ref[...] = jnp.maximum(outputs + t, 0.0)


def _logits_body(obj_ref, g_ref, b_ref, w_ref, cb_ref, out_ref):
    out_ref[...] = _dot(_ln(obj_ref[...], g_ref[...], b_ref[...]),
                        w_ref[...]) + cb_ref[...]


def _row2d(v):
    return v.reshape(1, -1)


def _call(body, grid, in_specs, out_specs, out_shape, *args):
    return pl.pallas_call(
        body, grid=grid, in_specs=in_specs, out_specs=out_specs,
        out_shape=out_shape)(*args)


def kernel(roi_features, union_features, pair_idxs, obj_labels, obj_embed, pos_embed, merge_W, merge_b, phr_W, phr_b, ws_W_0, ws_b_0, wo_W_0, wo_b_0, w_W_0, w_b_0, conv_W_0, conv_b_0, trans_W1_0, trans_b1_0, trans_W2_0, trans_b2_0, ln1_g_0, ln1_b_0, ln2_g_0, ln2_b_0, ws_W_1, ws_b_1, wo_W_1, wo_b_1, w_W_1, w_b_1, conv_W_1, conv_b_1, trans_W1_1, trans_b1_1, trans_W2_1, trans_b2_1, ln1_g_1, ln1_b_1, ln2_g_1, ln2_b_1, ln_g, ln_b, cls_W, cls_b):
    f = F32
    ep = jnp.concatenate([obj_embed, pos_embed,
                          jnp.zeros((N, EPD - 328), f)], axis=1)
    w_ep = jnp.concatenate([merge_W[OBJ:, :],
                            jnp.zeros((EPD - 328, H), f)], axis=0)
    p0 = pair_idxs[:, 0].astype(jnp.int32)
    p1 = pair_idxs[:, 1].astype(jnp.int32)
    p0p = jnp.concatenate([p0, jnp.zeros((Pp - P,), jnp.int32)])
    p1p = jnp.concatenate([p1, jnp.zeros((Pp - P,), jnp.int32)])
    valid = (jnp.arange(Pp) < P)
    key = jnp.where(valid, p0p * N + p1p, N * N + jnp.arange(Pp)).astype(jnp.int32)
    offd = ((p0p != p1p) & valid).astype(f)
    p0r, keyr, offdr = map(_row2d, (p0p, key, offd))
    unionp = jnp.concatenate([union_features, jnp.zeros((Pp - P, OBJ), f)], axis=0)

    full = lambda shp: pl.BlockSpec(shp, lambda i: (0,) * len(shp))
    rowN = pl.BlockSpec((TN, H), lambda i: (i, 0))
    rowP = pl.BlockSpec((TP, H), lambda i: (i, 0))
    vecP = pl.BlockSpec((1, TP), lambda i: (0, i))

    obj = _call(_merge_body, (GN,),
                [pl.BlockSpec((TN, OBJ), lambda i: (i, 0)),
                 pl.BlockSpec((TN, EPD), lambda i: (i, 0)),
                 full((OBJ, H)), full((EPD, H)), full((1, H))],
                rowN, jax.ShapeDtypeStruct((N, H), f),
                roi_features, ep, merge_W[:OBJ, :], w_ep, _row2d(merge_b))

    phr = _call(_phr_body, (GP,),
                [pl.BlockSpec((TP, OBJ), lambda i: (i, 0)),
                 full((OBJ, H)), full((1, H))],
                rowP, jax.ShapeDtypeStruct((Pp, H), f),
                unionp, phr_W, _row2d(phr_b))

    layers = [
        (ws_W_0, ws_b_0, wo_W_0, wo_b_0, w_W_0, w_b_0, conv_W_0, conv_b_0,
         trans_W1_0, trans_b1_0, trans_W2_0, trans_b2_0, ln1_g_0, ln1_b_0,
         ln2_g_0, ln2_b_0),
        (ws_W_1, ws_b_1, wo_W_1, wo_b_1, w_W_1, w_b_1, conv_W_1, conv_b_1,
         trans_W1_1, trans_b1_1, trans_W2_1, trans_b2_1, ln1_g_1, ln1_b_1,
         ln2_g_1, ln2_b_1),
    ]

    for (ws_W, ws_b, wo_W, wo_b, w_W, w_b, conv_W, conv_b, tW1, tb1, tW2,
         tb2, g1, b1, g2, b2) in layers:
        s, o, conv, csum = _call(
            _sov_body, (GN,),
            [rowN, full((H, H)), full((1, H)), full((H, H)), full((1, H)),
             full((H, H)), full((1, H)), full((1, H)), full((1, H))],
            [rowN, rowN, rowN, pl.BlockSpec((8, H), lambda i: (i, 0))],
            [jax.ShapeDtypeStruct((N, H), f)] * 3
            + [jax.ShapeDtypeStruct((GN * 8, H), f)],
            obj, ws_W, _row2d(ws_b), wo_W, _row2d(wo_b), conv_W,
            _row2d(conv_b), _row2d(g1), _row2d(b1))

        a16, cg = _sc_gather_a(s, o, conv, phr, w_W[:, 0], p0p, p1p)

        c, dv, wcg = _call(_paircoef_body, (GP,),
                           [full((1, Pp)), full((Pp, 16)), full((1, 128)),
                            full((1, Pp)), rowP],
                           [vecP, vecP, rowP],
                           [jax.ShapeDtypeStruct((1, Pp), f)] * 2
                           + [jax.ShapeDtypeStruct((Pp, H), f)],
                           keyr, a16,
                           jnp.broadcast_to(w_b.reshape(1, 1), (1, 128)),
                           offdr, cg)

        obj = _call(_ctx_body, (GN,),
                    [rowN, rowN, full((1, Pp)), full((1, Pp)),
                     full((1, Pp)), full((Pp, H)), full((GN * 8, H)),
                     full((1, H)), full((1, H)), full((H, 2 * H)),
                     full((1, 2 * H)), full((2 * H, H)), full((1, H))],
                    rowN, jax.ShapeDtypeStruct((N, H), f),
                    obj, conv, c, dv, p0r, wcg, csum,
                    _row2d(g2), _row2d(b2), tW1, _row2d(tb1), tW2,
                    _row2d(tb2))

    cls_Wp = jnp.concatenate([cls_W, jnp.zeros((H, NCLSP - 151), f)], axis=1)
    cls_bp = jnp.concatenate([cls_b, jnp.zeros((NCLSP - 151,), f)])
    logits = _call(_logits_body, (GN,),
                   [rowN, full((1, H)), full((1, H)), full((H, NCLSP)),
                    full((1, NCLSP))],
                   pl.BlockSpec((TN, NCLSP), lambda i: (i, 0)),
                   jax.ShapeDtypeStruct((N, NCLSP), f),
                   obj, _row2d(ln_g), _row2d(ln_b), cls_Wp, _row2d(cls_bp))
    return logits[:, :151]


# speedup vs baseline: 1.0700x; 1.0692x over previous
import functools

import jax
import jax.numpy as jnp
from jax import lax
from jax.experimental import pallas as pl
from jax.experimental.pallas import tpu as pltpu
from jax.experimental.pallas import tpu_sc as plsc

N = 2000
P = 4000
Pp = 4096
H = 512
OBJ = 4096
EPD = 384
NCLSP = 256
TN = 400
GN = 5
TP = 512
GP = 8
F32 = jnp.float32
NW = 32
BPW = Pp // NW


def _dot(a, b):
    return lax.dot_general(a, b, (((1,), (0,)), ((), ())),
                           precision=lax.Precision.DEFAULT,
                           preferred_element_type=F32)


def _ln(x, g, b):
    m = jnp.mean(x, axis=-1, keepdims=True)
    v = jnp.mean((x - m) ** 2, axis=-1, keepdims=True)
    return (x - m) / jnp.sqrt(v + 1e-5) * g + b


def _dotbf(a, b):
    return lax.dot_general(a.astype(jnp.bfloat16), b.astype(jnp.bfloat16),
                           (((1,), (0,)), ((), ())),
                           preferred_element_type=F32)


def _merge_body(roi_ref, ep_ref, w1_ref, w2_ref, b_ref, out_ref):
    acc = _dotbf(roi_ref[...], w1_ref[...])
    acc += _dotbf(ep_ref[...], w2_ref[...])
    out_ref[...] = acc + b_ref[...]


def _phr_body(u_ref, w_ref, b_ref, out_ref):
    out_ref[...] = _dotbf(u_ref[...], w_ref[...]) + b_ref[...]


def _sov_body(obj_ref, ws_ref, wsb_ref, wo_ref, wob_ref, cw_ref, cb_ref,
              g_ref, bb_ref, s_ref, o_ref, conv_ref, csum_ref):
    obj = obj_ref[...]
    s_ref[...] = _dotbf(obj, ws_ref[...]) + wsb_ref[...]
    o_ref[...] = _dotbf(obj, wo_ref[...]) + wob_ref[...]
    conv = jnp.maximum(_dotbf(_ln(obj, g_ref[...], bb_ref[...]), cw_ref[...])
                       + cb_ref[...], 0.0)
    conv_ref[...] = conv
    csum_ref[...] = jnp.broadcast_to(jnp.sum(conv, axis=0, keepdims=True),
                                     (8, H))


def _sc_gather_a(s, o, phr, wrow, p0, p1):
    CH = BPW // 2

    @functools.partial(
        pl.kernel,
        out_type=jax.ShapeDtypeStruct((Pp, 16), F32),
        mesh=plsc.VectorSubcoreMesh(core_axis_name="c", subcore_axis_name="s"),
        scratch_types=[pltpu.VMEM((BPW,), jnp.int32),
                       pltpu.VMEM((BPW,), jnp.int32),
                       pltpu.VMEM((H,), F32),
                       pltpu.VMEM((CH, H), F32),
                       pltpu.VMEM((CH, H), F32),
                       pltpu.VMEM((CH, H), F32),
                       pltpu.VMEM((CH, 16), F32),
                       pltpu.SemaphoreType.DMA,
                       pltpu.SemaphoreType.DMA,
                       pltpu.SemaphoreType.DMA],
    )
    def k(s_hbm, o_hbm, phr_hbm, w_hbm, p0_hbm, p1_hbm,
          a_hbm, i0_v, i1_v, wv, sgb, ogb, phb, avb, s0, s1, s2):
        wid = lax.axis_index("s") * 2 + lax.axis_index("c")
        base = wid * BPW
        pltpu.sync_copy(p0_hbm.at[pl.ds(base, BPW)], i0_v)
        pltpu.sync_copy(p1_hbm.at[pl.ds(base, BPW)], i1_v)
        pltpu.sync_copy(w_hbm, wv)
        for ch in range(2):
            g1 = pltpu.async_copy(
                s_hbm.at[i0_v.at[pl.ds(ch * CH, CH)]], sgb, s0)
            g2 = pltpu.async_copy(
                o_hbm.at[i1_v.at[pl.ds(ch * CH, CH)]], ogb, s1)
            g3 = pltpu.async_copy(
                phr_hbm.at[pl.ds(base + ch * CH, CH)], phb, s2)
            g1.wait()
            g2.wait()
            g3.wait()

            def body(r, carry):
                acc = jnp.zeros((16,), F32)
                for cc in range(H // 16):
                    hs = pl.ds(cc * 16, 16)
                    acc = acc + sgb[r, hs] * ogb[r, hs] * phb[r, hs] * wv[hs]
                avb[r, :] = acc
                return carry

            lax.fori_loop(0, CH, body, 0)
            pltpu.sync_copy(avb, a_hbm.at[pl.ds(base + ch * CH, CH)])

    return k(s, o, phr, wrow, p0, p1)


def _convg_body(p1_ref, conv_ref, out_ref):
    p1 = p1_ref[0, :]
    cols = lax.broadcasted_iota(jnp.int32, (TP, N), 1)
    oh1 = (p1[:, None] == cols).astype(F32)
    out_ref[...] = _dotbf(oh1, conv_ref[...])


def _paircoef_body(key_ref, a16_ref, wb_ref, offd_ref, cg_ref,
                   c_ref, d_ref, wcg_ref):
    i = pl.program_id(0)
    sl = pl.ds(i * TP, TP)
    kt = key_ref[0, sl]
    kf = key_ref[0, :]
    av = jnp.sum(a16_ref[...], axis=1) + wb_ref[0, 0]
    mm = kt[:, None] == kf[None, :]
    b = jnp.sum(jnp.where(mm, av[None, :], 0.0), axis=1)
    cnt = jnp.sum(mm.astype(F32), axis=1)
    invc = 1.0 / cnt
    offd = offd_ref[0, sl]
    c = offd * jnp.exp(b) * invc
    d = offd * invc
    c_ref[0, :] = c
    d_ref[0, :] = d
    wcg_ref[...] = cg_ref[...] * (c - d)[:, None]


def _ctx_body(obj_ref, conv_ref, c_ref, d_ref, p0_ref, wcg_ref,
              csum_ref, g2_ref, b2_ref, w1_ref, b1_ref, w2_ref, bb2_ref,
              out_ref):
    i = pl.program_id(0)
    rows = i * TN + lax.broadcasted_iota(jnp.int32, (TN, Pp), 0)
    p0 = p0_ref[0, :]
    rowmask = p0[None, :] == rows
    e = jnp.sum(jnp.where(rowmask, c_ref[0, :][None, :], 0.0), axis=1)
    dd = jnp.sum(jnp.where(rowmask, d_ref[0, :][None, :], 0.0), axis=1)
    ctxnum = _dotbf(rowmask.astype(F32), wcg_ref[...])
    z = ((N - 1) - dd) + e
    s_all = jnp.sum(csum_ref[...], axis=0, keepdims=True) * 0.125
    conv = conv_ref[...]
    context = ((s_all - conv) + ctxnum) / z[:, None]
    outputs = obj_ref[...] + context
    h = _ln(outputs, g2_ref[...], b2_ref[...])
    t = _dotbf(jnp.maximum(_dotbf(h, w1_ref[...]) + b1_ref[...], 0.0),
               w2_ref[...]) + bb2_ref[...]
    out_ref[...] = jnp.maximum(outputs + t, 0.0)


def _logits_body(obj_ref, g_ref, b_ref, w_ref, cb_ref, out_ref):
    out_ref[...] = _dot(_ln(obj_ref[...], g_ref[...], b_ref[...]),
                        w_ref[...]) + cb_ref[...]


def _row2d(v):
    return v.reshape(1, -1)


def _call(body, grid, in_specs, out_specs, out_shape, *args):
    return pl.pallas_call(
        body, grid=grid, in_specs=in_specs, out_specs=out_specs,
        out_shape=out_shape)(*args)


def kernel(roi_features, union_features, pair_idxs, obj_labels, obj_embed, pos_embed, merge_W, merge_b, phr_W, phr_b, ws_W_0, ws_b_0, wo_W_0, wo_b_0, w_W_0, w_b_0, conv_W_0, conv_b_0, trans_W1_0, trans_b1_0, trans_W2_0, trans_b2_0, ln1_g_0, ln1_b_0, ln2_g_0, ln2_b_0, ws_W_1, ws_b_1, wo_W_1, wo_b_1, w_W_1, w_b_1, conv_W_1, conv_b_1, trans_W1_1, trans_b1_1, trans_W2_1, trans_b2_1, ln1_g_1, ln1_b_1, ln2_g_1, ln2_b_1, ln_g, ln_b, cls_W, cls_b):
    f = F32
    ep = jnp.concatenate([obj_embed, pos_embed,
                          jnp.zeros((N, EPD - 328), f)], axis=1)
    w_ep = jnp.concatenate([merge_W[OBJ:, :],
                            jnp.zeros((EPD - 328, H), f)], axis=0)
    p0 = pair_idxs[:, 0].astype(jnp.int32)
    p1 = pair_idxs[:, 1].astype(jnp.int32)
    p0p = jnp.concatenate([p0, jnp.zeros((Pp - P,), jnp.int32)])
    p1p = jnp.concatenate([p1, jnp.zeros((Pp - P,), jnp.int32)])
    valid = (jnp.arange(Pp) < P)
    key = jnp.where(valid, p0p * N + p1p, N * N + jnp.arange(Pp)).astype(jnp.int32)
    offd = ((p0p != p1p) & valid).astype(f)
    p0r, p1r, keyr, offdr = map(_row2d, (p0p, p1p, key, offd))
    unionp = jnp.concatenate([union_features, jnp.zeros((Pp - P, OBJ), f)], axis=0)

    full = lambda shp: pl.BlockSpec(shp, lambda i: (0,) * len(shp))
    rowN = pl.BlockSpec((TN, H), lambda i: (i, 0))
    rowP = pl.BlockSpec((TP, H), lambda i: (i, 0))
    vecP = pl.BlockSpec((1, TP), lambda i: (0, i))

    obj = _call(_merge_body, (GN,),
                [pl.BlockSpec((TN, OBJ), lambda i: (i, 0)),
                 pl.BlockSpec((TN, EPD), lambda i: (i, 0)),
                 full((OBJ, H)), full((EPD, H)), full((1, H))],
                rowN, jax.ShapeDtypeStruct((N, H), f),
                roi_features, ep, merge_W[:OBJ, :], w_ep, _row2d(merge_b))

    phr = _call(_phr_body, (GP,),
                [pl.BlockSpec((TP, OBJ), lambda i: (i, 0)),
                 full((OBJ, H)), full((1, H))],
                rowP, jax.ShapeDtypeStruct((Pp, H), f),
                unionp, phr_W, _row2d(phr_b))

    layers = [
        (ws_W_0, ws_b_0, wo_W_0, wo_b_0, w_W_0, w_b_0, conv_W_0, conv_b_0,
         trans_W1_0, trans_b1_0, trans_W2_0, trans_b2_0, ln1_g_0, ln1_b_0,
         ln2_g_0, ln2_b_0),
        (ws_W_1, ws_b_1, wo_W_1, wo_b_1, w_W_1, w_b_1, conv_W_1, conv_b_1,
         trans_W1_1, trans_b1_1, trans_W2_1, trans_b2_1, ln1_g_1, ln1_b_1,
         ln2_g_1, ln2_b_1),
    ]

    for (ws_W, ws_b, wo_W, wo_b, w_W, w_b, conv_W, conv_b, tW1, tb1, tW2,
         tb2, g1, b1, g2, b2) in layers:
        s, o, conv, csum = _call(
            _sov_body, (GN,),
            [rowN, full((H, H)), full((1, H)), full((H, H)), full((1, H)),
             full((H, H)), full((1, H)), full((1, H)), full((1, H))],
            [rowN, rowN, rowN, pl.BlockSpec((8, H), lambda i: (i, 0))],
            [jax.ShapeDtypeStruct((N, H), f)] * 3
            + [jax.ShapeDtypeStruct((GN * 8, H), f)],
            obj, ws_W, _row2d(ws_b), wo_W, _row2d(wo_b), conv_W,
            _row2d(conv_b), _row2d(g1), _row2d(b1))

        a16 = _sc_gather_a(s, o, phr, w_W[:, 0], p0p, p1p)

        cg = _call(_convg_body, (GP,),
                   [pl.BlockSpec((1, TP), lambda i: (0, i)), full((N, H))],
                   rowP, jax.ShapeDtypeStruct((Pp, H), f),
                   p1r, conv)

        c, dv, wcg = _call(_paircoef_body, (GP,),
                           [full((1, Pp)), full((Pp, 16)), full((1, 128)),
                            full((1, Pp)), rowP],
                           [vecP, vecP, rowP],
                           [jax.ShapeDtypeStruct((1, Pp), f)] * 2
                           + [jax.ShapeDtypeStruct((Pp, H), f)],
                           keyr, a16,
                           jnp.broadcast_to(w_b.reshape(1, 1), (1, 128)),
                           offdr, cg)

        obj = _call(_ctx_body, (GN,),
                    [rowN, rowN, full((1, Pp)), full((1, Pp)),
                     full((1, Pp)), full((Pp, H)), full((GN * 8, H)),
                     full((1, H)), full((1, H)), full((H, 2 * H)),
                     full((1, 2 * H)), full((2 * H, H)), full((1, H))],
                    rowN, jax.ShapeDtypeStruct((N, H), f),
                    obj, conv, c, dv, p0r, wcg, csum,
                    _row2d(g2), _row2d(b2), tW1, _row2d(tb1), tW2,
                    _row2d(tb2))

    cls_Wp = jnp.concatenate([cls_W, jnp.zeros((H, NCLSP - 151), f)], axis=1)
    cls_bp = jnp.concatenate([cls_b, jnp.zeros((NCLSP - 151,), f)])
    logits = _call(_logits_body, (GN,),
                   [rowN, full((1, H)), full((1, H)), full((H, NCLSP)),
                    full((1, NCLSP))],
                   pl.BlockSpec((TN, NCLSP), lambda i: (i, 0)),
                   jax.ShapeDtypeStruct((N, NCLSP), f),
                   obj, _row2d(ln_g), _row2d(ln_b), cls_Wp, _row2d(cls_bp))
    return logits[:, :151]
